# Initial kernel scaffold; baseline (speedup 1.0000x reference)
import jax
import jax.numpy as jnp
from jax.experimental import pallas as pl


def kernel(token_ids, weight):
    raise NotImplementedError("write your pallas kernel here")


# speedup vs baseline: 1.4770x; 1.4770x over previous
import functools

import jax
import jax.numpy as jnp
from jax import lax
from jax.experimental import pallas as pl
from jax.experimental.pallas import tpu as pltpu
from jax.experimental.pallas import tpu_sc as plsc

NUM_EMB = 1_000_000
DIM = 32
B_TOTAL = 4096 * 200

_info = plsc.get_sparse_core_info()
NC = _info.num_cores
NS = _info.num_subcores
NW = NC * NS
B_PER_W = B_TOTAL // NW
CHUNK = 1600
NCHUNK = B_PER_W // CHUNK


def _emb_body(idx_hbm, tbl_hbm, out_hbm, idx_v, rows_v, sem):
    wid = lax.axis_index("s") * NC + lax.axis_index("c")
    base = wid * B_PER_W

    def chunk_body(c, carry):
        off = base + c * CHUNK
        pltpu.sync_copy(idx_hbm.at[pl.ds(off, CHUNK)], idx_v)
        pltpu.async_copy(tbl_hbm.at[idx_v], rows_v, sem).wait()
        pltpu.sync_copy(rows_v, out_hbm.at[pl.ds(off, CHUNK)])
        return carry

    lax.fori_loop(0, NCHUNK, chunk_body, 0)


_emb_call = functools.partial(
    pl.kernel,
    mesh=plsc.VectorSubcoreMesh(core_axis_name="c", subcore_axis_name="s"),
    out_type=jax.ShapeDtypeStruct((B_TOTAL, DIM), jnp.float32),
    scratch_types=[
        pltpu.VMEM((CHUNK,), jnp.int32),
        pltpu.VMEM((CHUNK, DIM), jnp.float32),
        pltpu.SemaphoreType.DMA,
    ],
    compiler_params=pltpu.CompilerParams(use_tc_tiling_on_sc=False),
)(_emb_body)


@jax.jit
def kernel(token_ids, weight):
    idx = token_ids.reshape(-1).astype(jnp.int32)
    out = _emb_call(idx, weight)
    return out.reshape(token_ids.shape + (DIM,))


# speedup vs baseline: 1.5004x; 1.0158x over previous
import functools

import jax
import jax.numpy as jnp
from jax import lax
from jax.experimental import pallas as pl
from jax.experimental.pallas import tpu as pltpu
from jax.experimental.pallas import tpu_sc as plsc

NUM_EMB = 1_000_000
DIM = 32
B_TOTAL = 4096 * 200

_info = plsc.get_sparse_core_info()
NC = _info.num_cores
NS = _info.num_subcores
NW = NC * NS
B_PER_W = B_TOTAL // NW
CHUNK = 1600
NCHUNK = B_PER_W // CHUNK


def _emb_body(idx_hbm, tbl_hbm, out_hbm,
              idx_v0, idx_v1, rows_v0, rows_v1,
              si0, si1, sg0, sg1, so0, so1):
    idx_v = (idx_v0, idx_v1)
    rows_v = (rows_v0, rows_v1)
    si = (si0, si1)
    sg = (sg0, sg1)
    so = (so0, so1)

    wid = lax.axis_index("s") * NC + lax.axis_index("c")
    base = wid * B_PER_W

    def idx_slice(c):
        return idx_hbm.at[pl.ds(base + c * CHUNK, CHUNK)]

    def out_slice(c):
        return out_hbm.at[pl.ds(base + c * CHUNK, CHUNK)]

    pltpu.async_copy(idx_slice(0), idx_v[0], si[0])
    pltpu.async_copy(idx_slice(1), idx_v[1], si[1])

    for c in range(NCHUNK):
        b = c % 2
        if c >= 2:
            pltpu.make_async_copy(rows_v[b], out_slice(c - 2), so[b]).wait()
        pltpu.make_async_copy(idx_slice(c), idx_v[b], si[b]).wait()
        pltpu.async_copy(tbl_hbm.at[idx_v[b]], rows_v[b], sg[b])
        if c >= 1:
            b1 = (c - 1) % 2
            pltpu.make_async_copy(tbl_hbm.at[idx_v[b1]], rows_v[b1], sg[b1]).wait()
            pltpu.async_copy(rows_v[b1], out_slice(c - 1), so[b1])
            if c + 1 < NCHUNK:
                pltpu.async_copy(idx_slice(c + 1), idx_v[b1], si[b1])

    bl = (NCHUNK - 1) % 2
    pltpu.make_async_copy(tbl_hbm.at[idx_v[bl]], rows_v[bl], sg[bl]).wait()
    pltpu.async_copy(rows_v[bl], out_slice(NCHUNK - 1), so[bl])
    pltpu.make_async_copy(rows_v[1 - bl], out_slice(NCHUNK - 2), so[1 - bl]).wait()
    pltpu.make_async_copy(rows_v[bl], out_slice(NCHUNK - 1), so[bl]).wait()


_emb_call = functools.partial(
    pl.kernel,
    mesh=plsc.VectorSubcoreMesh(core_axis_name="c", subcore_axis_name="s"),
    out_type=jax.ShapeDtypeStruct((B_TOTAL, DIM), jnp.float32),
    scratch_types=[
        pltpu.VMEM((CHUNK,), jnp.int32),
        pltpu.VMEM((CHUNK,), jnp.int32),
        pltpu.VMEM((CHUNK, DIM), jnp.float32),
        pltpu.VMEM((CHUNK, DIM), jnp.float32),
        pltpu.SemaphoreType.DMA,
        pltpu.SemaphoreType.DMA,
        pltpu.SemaphoreType.DMA,
        pltpu.SemaphoreType.DMA,
        pltpu.SemaphoreType.DMA,
        pltpu.SemaphoreType.DMA,
    ],
    compiler_params=pltpu.CompilerParams(use_tc_tiling_on_sc=False),
)(_emb_body)


@jax.jit
def kernel(token_ids, weight):
    idx = token_ids.reshape(-1).astype(jnp.int32)
    out = _emb_call(idx, weight)
    return out.reshape(token_ids.shape + (DIM,))
